# Initial kernel scaffold; baseline (speedup 1.0000x reference)
import jax
import jax.numpy as jnp
from jax.experimental import pallas as pl


def kernel(user_emb, item_emb, user_emb_implict, item_emb_implict, adj_row, adj_col, adj_val, adj_imp_row, adj_imp_col, adj_imp_val):
    raise NotImplementedError("write your pallas kernel here")


# speedup vs baseline: 2.5503x; 2.5503x over previous
import functools

import jax
import jax.numpy as jnp
from jax import lax
from jax.experimental import pallas as pl
from jax.experimental.pallas import tpu as pltpu
from jax.experimental.pallas import tpu_sc as plsc

USERS = 4000
ITEMS = 6000
NN = USERS + ITEMS
DD = 128
EE = 320000
NT = 16
CH = 128
NCHP = 2560
EEP = NCHP * CH
CPT = NCHP // NT
GRP = 16
NGRP = CPT // GRP
GE = GRP * CH
NNP = 10240
RTP = NNP // NT
RC = 128
NRC = RTP // RC
SB = 32
LANES = 16
NG = DD // LANES


def _gcn_body(ego, col, rowr, val, out, y1, y2,
              acc, colg, rowg, valg, rbuf, sbuf, sem):
    c = lax.axis_index("c")
    tile = lax.axis_index("s")
    rbase = pl.multiple_of(c * NNP + tile * RTP, RTP)
    cbase = pl.multiple_of(c * NCHP + tile * CPT, CPT)

    def _zero_sbuf(i, _):
        for g in range(NG):
            sbuf[i, pl.ds(g * LANES, LANES)] = jnp.zeros((LANES,), jnp.float32)
        return 0
    lax.fori_loop(0, SB, _zero_sbuf, 0)
    for k in range(RTP // SB):
        pltpu.sync_copy(
            sbuf, acc.at[pl.ds(pl.multiple_of(tile * RTP + k * SB, SB), SB)])
    plsc.subcore_barrier()

    def spmm(src):
        def group_body(grp, _):
            gchunk = pl.multiple_of(cbase + grp * GRP, GRP)
            gedge = pl.multiple_of(gchunk * CH, GE)
            pltpu.sync_copy(col.at[pl.ds(gedge, GE)], colg)
            pltpu.sync_copy(val.at[pl.ds(gedge, GE)], valg)
            pltpu.sync_copy(rowr.at[pl.ds(gchunk, GRP)], rowg)

            def chunk_body(j, _):
                off = pl.multiple_of(j * CH, CH)
                pltpu.async_copy(src.at[colg.at[pl.ds(off, CH)]],
                                 rbuf, sem).wait()

                def mul_body(eb, _):
                    vv = valg[pl.ds(j * CH + eb * LANES, LANES)]
                    for i in range(LANES):
                        sv = vv[i]
                        e = eb * LANES + i
                        for g in range(NG):
                            sl = pl.ds(g * LANES, LANES)
                            rbuf[e, sl] = rbuf[e, sl] * sv
                    return 0
                lax.fori_loop(0, CH // LANES, mul_body, 0)

                pltpu.sync_copy(rbuf, acc.at[rowg.at[j]], add=True)
                return 0
            lax.fori_loop(0, GRP, chunk_body, 0)
            return 0
        lax.fori_loop(0, NGRP, group_body, 0)

    def drain(dst):
        for k in range(NRC):
            lro = pl.multiple_of(tile * RTP + k * RC, RC)
            ro = pl.multiple_of(rbase + k * RC, RC)
            pltpu.sync_copy(acc.at[pl.ds(lro, RC)], rbuf)
            pltpu.sync_copy(rbuf, dst.at[pl.ds(ro, RC)])
            for m in range(RC // SB):
                pltpu.sync_copy(
                    sbuf, acc.at[pl.ds(pl.multiple_of(lro + m * SB, SB), SB)])

    def final():
        for k in range(NRC):
            lro = pl.multiple_of(tile * RTP + k * RC, RC)
            ro = pl.multiple_of(rbase + k * RC, RC)
            pltpu.sync_copy(acc.at[pl.ds(lro, RC)], rbuf)
            for srcref in (ego, y1, y2):
                for m in range(RC // SB):
                    pltpu.sync_copy(
                        srcref.at[pl.ds(pl.multiple_of(ro + m * SB, SB), SB)],
                        sbuf)

                    def add_body(i, _):
                        for g in range(NG):
                            sl = pl.ds(g * LANES, LANES)
                            rbuf[m * SB + i, sl] = (rbuf[m * SB + i, sl]
                                                    + sbuf[i, sl])
                        return 0
                    lax.fori_loop(0, SB, add_body, 0)

            def scale_body(i, _):
                for g in range(NG):
                    sl = pl.ds(g * LANES, LANES)
                    rbuf[i, sl] = rbuf[i, sl] * 0.25
                return 0
            lax.fori_loop(0, RC, scale_body, 0)
            pltpu.sync_copy(rbuf, out.at[pl.ds(ro, RC)])

    spmm(ego)
    plsc.subcore_barrier()
    drain(y1)
    plsc.subcore_barrier()
    spmm(y1)
    plsc.subcore_barrier()
    drain(y2)
    plsc.subcore_barrier()
    spmm(y2)
    plsc.subcore_barrier()
    final()


_gcn = functools.partial(
    pl.kernel,
    out_type=[jax.ShapeDtypeStruct((2 * NNP, DD), jnp.float32)] * 3,
    mesh=plsc.VectorSubcoreMesh(core_axis_name="c", subcore_axis_name="s"),
    scratch_types=[
        pltpu.VMEM_SHARED((NNP, DD), jnp.float32),
        pltpu.VMEM((GE,), jnp.int32),
        pltpu.VMEM((GRP, CH), jnp.int32),
        pltpu.VMEM((GE,), jnp.float32),
        pltpu.VMEM((RC, DD), jnp.float32),
        pltpu.VMEM((SB, DD), jnp.float32),
        pltpu.SemaphoreType.DMA,
    ],
)(_gcn_body)


def kernel(user_emb, item_emb, user_emb_implict, item_emb_implict,
           adj_row, adj_col, adj_val, adj_imp_row, adj_imp_col, adj_imp_val):
    npad = NNP - NN
    epad = EEP - EE
    ego = jnp.concatenate([
        jnp.pad(jnp.concatenate([user_emb, item_emb], axis=0),
                ((0, npad), (0, 0))),
        jnp.pad(jnp.concatenate([user_emb_implict, item_emb_implict], axis=0),
                ((0, npad), (0, 0))),
    ])
    col = jnp.concatenate([jnp.pad(adj_col, (0, epad)),
                           jnp.pad(adj_imp_col, (0, epad)) + NNP])
    row = jnp.concatenate([jnp.pad(adj_row, (0, epad)),
                           jnp.pad(adj_imp_row, (0, epad))]).reshape(
                               2 * NCHP, CH)
    val = jnp.concatenate([jnp.pad(adj_val, (0, epad)),
                           jnp.pad(adj_imp_val, (0, epad))])
    out, _, _ = _gcn(ego, col, row, val)
    return (out[:USERS], out[USERS:NN],
            out[NNP:NNP + USERS], out[NNP + USERS:NNP + NN])


# speedup vs baseline: 6.8997x; 2.7054x over previous
import functools

import jax
import jax.numpy as jnp
from jax import lax
from jax.experimental import pallas as pl
from jax.experimental.pallas import tpu as pltpu
from jax.experimental.pallas import tpu_sc as plsc

USERS = 4000
ITEMS = 6000
NN = USERS + ITEMS
DD = 128
EE = 320000
NT = 16
CH = 128
NCHP = 2560
EEP = NCHP * CH
CPT = NCHP // NT
GRP = 16
NGRP = CPT // GRP
GE = GRP * CH
NNP = 10240
RTP = NNP // NT
RC = 128
NRC = RTP // RC
SB = 32
LANES = 16
NG = DD // LANES


def _gcn_body(ego, col, rowr, val, out, y1, y2,
              acc, colg, rowg, valg, rbuf, rbuf1, sbuf, sem, sem1):
    c = lax.axis_index("c")
    tile = lax.axis_index("s")
    rbase = pl.multiple_of(c * NNP + tile * RTP, RTP)
    cbase = pl.multiple_of(c * NCHP + tile * CPT, CPT)

    def _zero_sbuf(i, _):
        for g in range(NG):
            sbuf[i, pl.ds(g * LANES, LANES)] = jnp.zeros((LANES,), jnp.float32)
        return 0
    lax.fori_loop(0, SB, _zero_sbuf, 0)
    for k in range(RTP // SB):
        pltpu.sync_copy(
            sbuf, acc.at[pl.ds(pl.multiple_of(tile * RTP + k * SB, SB), SB)])
    plsc.subcore_barrier()

    def spmm(src):
        def start(j, buf, s):
            off = pl.multiple_of(j * CH, CH)
            pltpu.async_copy(src.at[colg.at[pl.ds(off, CH)]], buf, s)

        def wait(buf, s):
            pltpu.make_async_copy(src.at[pl.ds(0, CH)], buf, s).wait()

        def process(j, buf):
            def mul_body(eb, _):
                vv = valg[pl.ds(j * CH + eb * LANES, LANES)]
                for i in range(LANES):
                    sv = vv[i]
                    e = eb * LANES + i
                    for g in range(NG):
                        sl = pl.ds(g * LANES, LANES)
                        buf[e, sl] = buf[e, sl] * sv
                return 0
            lax.fori_loop(0, CH // LANES, mul_body, 0)
            pltpu.sync_copy(buf, acc.at[rowg.at[j]], add=True)

        def group_body(grp, _):
            gchunk = pl.multiple_of(cbase + grp * GRP, GRP)
            gedge = pl.multiple_of(gchunk * CH, GE)
            pltpu.sync_copy(col.at[pl.ds(gedge, GE)], colg)
            pltpu.sync_copy(val.at[pl.ds(gedge, GE)], valg)
            pltpu.sync_copy(rowr.at[pl.ds(gchunk, GRP)], rowg)

            start(0, rbuf, sem)

            def pair_body(k, _):
                j = k * 2
                start(j + 1, rbuf1, sem1)
                wait(rbuf, sem)
                process(j, rbuf)
                start(j + 2, rbuf, sem)
                wait(rbuf1, sem1)
                process(j + 1, rbuf1)
                return 0
            lax.fori_loop(0, GRP // 2 - 1, pair_body, 0)

            start(GRP - 1, rbuf1, sem1)
            wait(rbuf, sem)
            process(GRP - 2, rbuf)
            wait(rbuf1, sem1)
            process(GRP - 1, rbuf1)
            return 0
        lax.fori_loop(0, NGRP, group_body, 0)

    def drain(dst):
        for k in range(NRC):
            lro = pl.multiple_of(tile * RTP + k * RC, RC)
            ro = pl.multiple_of(rbase + k * RC, RC)
            pltpu.sync_copy(acc.at[pl.ds(lro, RC)], rbuf)
            pltpu.sync_copy(rbuf, dst.at[pl.ds(ro, RC)])
            for m in range(RC // SB):
                pltpu.sync_copy(
                    sbuf, acc.at[pl.ds(pl.multiple_of(lro + m * SB, SB), SB)])

    def final():
        for k in range(NRC):
            lro = pl.multiple_of(tile * RTP + k * RC, RC)
            ro = pl.multiple_of(rbase + k * RC, RC)
            pltpu.sync_copy(acc.at[pl.ds(lro, RC)], rbuf)
            for srcref in (ego, y1, y2):
                for m in range(RC // SB):
                    pltpu.sync_copy(
                        srcref.at[pl.ds(pl.multiple_of(ro + m * SB, SB), SB)],
                        sbuf)

                    def add_body(i, _):
                        for g in range(NG):
                            sl = pl.ds(g * LANES, LANES)
                            rbuf[m * SB + i, sl] = (rbuf[m * SB + i, sl]
                                                    + sbuf[i, sl])
                        return 0
                    lax.fori_loop(0, SB, add_body, 0)

            def scale_body(i, _):
                for g in range(NG):
                    sl = pl.ds(g * LANES, LANES)
                    rbuf[i, sl] = rbuf[i, sl] * 0.25
                return 0
            lax.fori_loop(0, RC, scale_body, 0)
            pltpu.sync_copy(rbuf, out.at[pl.ds(ro, RC)])

    spmm(ego)
    plsc.subcore_barrier()
    drain(y1)
    plsc.subcore_barrier()
    spmm(y1)
    plsc.subcore_barrier()
    drain(y2)
    plsc.subcore_barrier()
    spmm(y2)
    plsc.subcore_barrier()
    final()


_gcn = functools.partial(
    pl.kernel,
    out_type=[jax.ShapeDtypeStruct((2 * NNP, DD), jnp.float32)] * 3,
    mesh=plsc.VectorSubcoreMesh(core_axis_name="c", subcore_axis_name="s"),
    scratch_types=[
        pltpu.VMEM_SHARED((NNP, DD), jnp.float32),
        pltpu.VMEM((GE,), jnp.int32),
        pltpu.VMEM((GRP, CH), jnp.int32),
        pltpu.VMEM((GE,), jnp.float32),
        pltpu.VMEM((RC, DD), jnp.float32),
        pltpu.VMEM((RC, DD), jnp.float32),
        pltpu.VMEM((SB, DD), jnp.float32),
        pltpu.SemaphoreType.DMA,
        pltpu.SemaphoreType.DMA,
    ],
)(_gcn_body)


def kernel(user_emb, item_emb, user_emb_implict, item_emb_implict,
           adj_row, adj_col, adj_val, adj_imp_row, adj_imp_col, adj_imp_val):
    npad = NNP - NN
    epad = EEP - EE
    ego = jnp.concatenate([
        jnp.pad(jnp.concatenate([user_emb, item_emb], axis=0),
                ((0, npad), (0, 0))),
        jnp.pad(jnp.concatenate([user_emb_implict, item_emb_implict], axis=0),
                ((0, npad), (0, 0))),
    ])
    pad_idx = jnp.arange(epad, dtype=jnp.int32) % NN
    col = jnp.concatenate([adj_col, pad_idx,
                           adj_imp_col + NNP, pad_idx + NNP])
    row = jnp.concatenate([adj_row, pad_idx,
                           adj_imp_row, pad_idx]).reshape(2 * NCHP, CH)
    val = jnp.concatenate([jnp.pad(adj_val, (0, epad)),
                           jnp.pad(adj_imp_val, (0, epad))])
    out, _, _ = _gcn(ego, col, row, val)
    return (out[:USERS], out[USERS:NN],
            out[NNP:NNP + USERS], out[NNP + USERS:NNP + NN])


# speedup vs baseline: 7.2330x; 1.0483x over previous
import functools

import jax
import jax.numpy as jnp
from jax import lax
from jax.experimental import pallas as pl
from jax.experimental.pallas import tpu as pltpu
from jax.experimental.pallas import tpu_sc as plsc

USERS = 4000
ITEMS = 6000
NN = USERS + ITEMS
DD = 128
EE = 320000
NT = 16
CH = 64
NCHP = 5120
EEP = NCHP * CH
CPT = NCHP // NT
GRP = 32
NGRP = CPT // GRP
GE = GRP * CH
NNP = 10240
RTP = NNP // NT
RC = 64
NRC = RTP // RC
LANES = 16
NG = DD // LANES


def _gcn_body(ego, col, rowr, val, out, y1, y2,
              acc, colg, rowg, valg, b0, b1, b2, b3,
              gs0, gs1, gs2, gs3, ss0, ss1, ss2, ss3):
    c = lax.axis_index("c")
    tile = lax.axis_index("s")
    rbase = pl.multiple_of(c * NNP + tile * RTP, RTP)
    cbase = pl.multiple_of(c * NCHP + tile * CPT, CPT)
    bufs = (b0, b1, b2, b3)
    gsems = (gs0, gs1, gs2, gs3)
    ssems = (ss0, ss1, ss2, ss3)

    def zero_buf(b):
        def zb(i, _):
            for g in range(NG):
                b[i, pl.ds(g * LANES, LANES)] = jnp.zeros((LANES,),
                                                          jnp.float32)
            return 0
        lax.fori_loop(0, RC, zb, 0)

    zero_buf(b3)
    for k in range(NRC):
        pltpu.sync_copy(
            b3, acc.at[pl.ds(pl.multiple_of(tile * RTP + k * RC, RC), RC)])
    plsc.subcore_barrier()

    def spmm(src):
        def start_g(ck, b, gs):
            off = pl.multiple_of(ck * CH, CH)
            pltpu.async_copy(src.at[colg.at[pl.ds(off, CH)]], b, gs)

        def wait_g(b, gs):
            pltpu.make_async_copy(src.at[pl.ds(0, CH)], b, gs).wait()

        def start_s(ck, b, ss):
            pltpu.async_copy(b, acc.at[rowg.at[ck]], ss, add=True)

        def wait_s(b, ss):
            pltpu.make_async_copy(b, acc.at[pl.ds(0, CH)], ss).wait()

        def mult(ck, b):
            def mul_body(eb, _):
                vv = valg[pl.ds(ck * CH + eb * LANES, LANES)]
                for i in range(LANES):
                    sv = vv[i]
                    e = eb * LANES + i
                    for g in range(NG):
                        sl = pl.ds(g * LANES, LANES)
                        b[e, sl] = b[e, sl] * sv
                return 0
            lax.fori_loop(0, CH // LANES, mul_body, 0)

        def group_body(grp, _):
            gchunk = pl.multiple_of(cbase + grp * GRP, GRP)
            gedge = pl.multiple_of(gchunk * CH, GE)
            pltpu.sync_copy(col.at[pl.ds(gedge, GE)], colg)
            pltpu.sync_copy(val.at[pl.ds(gedge, GE)], valg)
            pltpu.sync_copy(rowr.at[pl.ds(gchunk, GRP)], rowg)

            start_g(0, bufs[0], gsems[0])
            start_g(1, bufs[1], gsems[1])

            def quad_body(k, _):
                for j in range(4):
                    ck = k * 4 + j
                    nb = (j + 2) % 4
                    wait_g(bufs[j], gsems[j])
                    mult(ck, bufs[j])
                    start_s(ck, bufs[j], ssems[j])

                    @pl.when(ck >= 2)
                    def _():
                        wait_s(bufs[nb], ssems[nb])

                    @pl.when(ck + 2 < GRP)
                    def _():
                        start_g(ck + 2, bufs[nb], gsems[nb])
                return 0
            lax.fori_loop(0, GRP // 4, quad_body, 0)

            wait_s(bufs[2], ssems[2])
            wait_s(bufs[3], ssems[3])
            return 0
        lax.fori_loop(0, NGRP, group_body, 0)

    def drain(dst):
        zero_buf(b1)
        for k in range(NRC):
            lro = pl.multiple_of(tile * RTP + k * RC, RC)
            ro = pl.multiple_of(rbase + k * RC, RC)
            pltpu.sync_copy(acc.at[pl.ds(lro, RC)], b0)
            pltpu.sync_copy(b0, dst.at[pl.ds(ro, RC)])
            pltpu.sync_copy(b1, acc.at[pl.ds(lro, RC)])

    def final():
        for k in range(NRC):
            lro = pl.multiple_of(tile * RTP + k * RC, RC)
            ro = pl.multiple_of(rbase + k * RC, RC)
            pltpu.sync_copy(acc.at[pl.ds(lro, RC)], b0)
            for srcref in (ego, y1, y2):
                pltpu.sync_copy(srcref.at[pl.ds(ro, RC)], b1)

                def add_body(i, _):
                    for g in range(NG):
                        sl = pl.ds(g * LANES, LANES)
                        b0[i, sl] = b0[i, sl] + b1[i, sl]
                    return 0
                lax.fori_loop(0, RC, add_body, 0)

            def scale_body(i, _):
                for g in range(NG):
                    sl = pl.ds(g * LANES, LANES)
                    b0[i, sl] = b0[i, sl] * 0.25
                return 0
            lax.fori_loop(0, RC, scale_body, 0)
            pltpu.sync_copy(b0, out.at[pl.ds(ro, RC)])

    spmm(ego)
    plsc.subcore_barrier()
    drain(y1)
    plsc.subcore_barrier()
    spmm(y1)
    plsc.subcore_barrier()
    drain(y2)
    plsc.subcore_barrier()
    spmm(y2)
    plsc.subcore_barrier()
    final()


_gcn = functools.partial(
    pl.kernel,
    out_type=[jax.ShapeDtypeStruct((2 * NNP, DD), jnp.float32)] * 3,
    mesh=plsc.VectorSubcoreMesh(core_axis_name="c", subcore_axis_name="s"),
    scratch_types=[
        pltpu.VMEM_SHARED((NNP, DD), jnp.float32),
        pltpu.VMEM((GE,), jnp.int32),
        pltpu.VMEM((GRP, CH), jnp.int32),
        pltpu.VMEM((GE,), jnp.float32),
        pltpu.VMEM((RC, DD), jnp.float32),
        pltpu.VMEM((RC, DD), jnp.float32),
        pltpu.VMEM((RC, DD), jnp.float32),
        pltpu.VMEM((RC, DD), jnp.float32),
        pltpu.SemaphoreType.DMA,
        pltpu.SemaphoreType.DMA,
        pltpu.SemaphoreType.DMA,
        pltpu.SemaphoreType.DMA,
        pltpu.SemaphoreType.DMA,
        pltpu.SemaphoreType.DMA,
        pltpu.SemaphoreType.DMA,
        pltpu.SemaphoreType.DMA,
    ],
)(_gcn_body)


def kernel(user_emb, item_emb, user_emb_implict, item_emb_implict,
           adj_row, adj_col, adj_val, adj_imp_row, adj_imp_col, adj_imp_val):
    npad = NNP - NN
    epad = EEP - EE
    ego = jnp.concatenate([
        jnp.pad(jnp.concatenate([user_emb, item_emb], axis=0),
                ((0, npad), (0, 0))),
        jnp.pad(jnp.concatenate([user_emb_implict, item_emb_implict], axis=0),
                ((0, npad), (0, 0))),
    ])
    pad_idx = jnp.arange(epad, dtype=jnp.int32) % NN
    col = jnp.concatenate([adj_col, pad_idx,
                           adj_imp_col + NNP, pad_idx + NNP])
    row = jnp.concatenate([adj_row, pad_idx,
                           adj_imp_row, pad_idx]).reshape(2 * NCHP, CH)
    val = jnp.concatenate([jnp.pad(adj_val, (0, epad)),
                           jnp.pad(adj_imp_val, (0, epad))])
    out, _, _ = _gcn(ego, col, row, val)
    return (out[:USERS], out[USERS:NN],
            out[NNP:NNP + USERS], out[NNP + USERS:NNP + NN])


# speedup vs baseline: 7.9491x; 1.0990x over previous
import functools

import jax
import jax.numpy as jnp
from jax import lax
from jax.experimental import pallas as pl
from jax.experimental.pallas import tpu as pltpu
from jax.experimental.pallas import tpu_sc as plsc

USERS = 4000
ITEMS = 6000
NN = USERS + ITEMS
DD = 128
EE = 320000
NT = 16
CH = 64
NCHP = 5120
EEP = NCHP * CH
CPT = NCHP // NT
GRP = 32
NGRP = CPT // GRP
GE = GRP * CH
NNP = 10240
RTP = NNP // NT
RC = 64
NRC = RTP // RC
LANES = 16
NG = DD // LANES


def _gcn_body(ego, col, rowr, val, out, y1, y2,
              acc, colg, rowg, valg, b0, b1, b2, b3,
              gs0, gs1, gs2, gs3, ss0, ss1, ss2, ss3, ms0, ms1):
    c = lax.axis_index("c")
    tile = lax.axis_index("s")
    rbase = pl.multiple_of(c * NNP + tile * RTP, RTP)
    cbase = pl.multiple_of(c * NCHP + tile * CPT, CPT)
    bufs = (b0, b1, b2, b3)
    gsems = (gs0, gs1, gs2, gs3)
    ssems = (ss0, ss1, ss2, ss3)

    def zero_buf(b):
        def zb(i, _):
            for g in range(NG):
                b[i, pl.ds(g * LANES, LANES)] = jnp.zeros((LANES,),
                                                          jnp.float32)
            return 0
        lax.fori_loop(0, RC, zb, 0)

    def lrow(k):
        return pl.multiple_of(tile * RTP + k * RC, RC)

    def grow(k):
        return pl.multiple_of(rbase + k * RC, RC)

    zero_buf(b3)
    for k in range(NRC):
        pltpu.async_copy(b3, acc.at[pl.ds(lrow(k), RC)], gs2)
    for k in range(NRC):
        pltpu.make_async_copy(b3, acc.at[pl.ds(0, RC)], gs2).wait()
    plsc.subcore_barrier()

    def spmm(src):
        def start_g(ck, b, gs, boff):
            pltpu.async_copy(src.at[colg.at[pl.ds(boff + ck * CH, CH)]],
                             b, gs)

        def wait_g(b, gs):
            pltpu.make_async_copy(src.at[pl.ds(0, CH)], b, gs).wait()

        def start_s(ck, b, ss, goff):
            pltpu.async_copy(b, acc.at[rowg.at[goff + ck]], ss, add=True)

        def wait_s(b, ss):
            pltpu.make_async_copy(b, acc.at[pl.ds(0, CH)], ss).wait()

        def mult(ck, b, boff):
            def mul_body(eb, _):
                vv = valg[pl.ds(boff + ck * CH + eb * LANES, LANES)]
                for i in range(LANES):
                    sv = vv[i]
                    e = eb * LANES + i
                    for g in range(NG):
                        sl = pl.ds(g * LANES, LANES)
                        b[e, sl] = b[e, sl] * sv
                return 0
            lax.fori_loop(0, CH // LANES, mul_body, 0)

        def load_meta(grp, boff, goff, sem):
            gchunk = pl.multiple_of(cbase + grp * GRP, GRP)
            gedge = pl.multiple_of(gchunk * CH, GE)
            pltpu.async_copy(col.at[pl.ds(gedge, GE)],
                             colg.at[pl.ds(boff, GE)], sem)
            pltpu.async_copy(val.at[pl.ds(gedge, GE)],
                             valg.at[pl.ds(boff, GE)], sem)
            pltpu.async_copy(rowr.at[pl.ds(gchunk, GRP)],
                             rowg.at[pl.ds(goff, GRP)], sem)

        def wait_meta(sem):
            pltpu.make_async_copy(col.at[pl.ds(0, GE)],
                                  colg.at[pl.ds(0, GE)], sem).wait()
            pltpu.make_async_copy(val.at[pl.ds(0, GE)],
                                  valg.at[pl.ds(0, GE)], sem).wait()
            pltpu.make_async_copy(rowr.at[pl.ds(0, GRP)],
                                  rowg.at[pl.ds(0, GRP)], sem).wait()

        load_meta(0, 0, 0, ms0)
        wait_meta(ms0)
        load_meta(1, GE, GRP, ms1)

        def group_body(grp, _):
            parity = lax.rem(grp, 2)
            boff = pl.multiple_of(parity * GE, GE)
            goff = pl.multiple_of(parity * GRP, GRP)

            start_g(0, bufs[0], gsems[0], boff)
            start_g(1, bufs[1], gsems[1], boff)

            def quad_body(k, _):
                for j in range(4):
                    ck = k * 4 + j
                    nb = (j + 2) % 4
                    wait_g(bufs[j], gsems[j])
                    mult(ck, bufs[j], boff)
                    start_s(ck, bufs[j], ssems[j], goff)

                    @pl.when(ck >= 2)
                    def _():
                        wait_s(bufs[nb], ssems[nb])

                    @pl.when(ck + 2 < GRP)
                    def _():
                        start_g(ck + 2, bufs[nb], gsems[nb], boff)
                return 0
            lax.fori_loop(0, GRP // 4, quad_body, 0)

            wait_s(bufs[2], ssems[2])
            wait_s(bufs[3], ssems[3])

            @pl.when(grp + 2 < NGRP)
            def _():
                @pl.when(parity == 0)
                def _():
                    load_meta(grp + 2, 0, 0, ms0)

                @pl.when(parity == 1)
                def _():
                    load_meta(grp + 2, GE, GRP, ms1)

            @pl.when(grp + 1 < NGRP)
            def _():
                @pl.when(parity == 0)
                def _():
                    wait_meta(ms1)

                @pl.when(parity == 1)
                def _():
                    wait_meta(ms0)
            return 0
        lax.fori_loop(0, NGRP, group_body, 0)

    def drain(dst):
        zero_buf(b1)

        def rd(k, b, sem):
            pltpu.async_copy(acc.at[pl.ds(lrow(k), RC)], b, sem)

        def wr(k, b, sem):
            pltpu.async_copy(b, dst.at[pl.ds(grow(k), RC)], sem)

        def zr(k):
            pltpu.async_copy(b1, acc.at[pl.ds(lrow(k), RC)], gs2)

        def wait_rd(b, sem):
            pltpu.make_async_copy(acc.at[pl.ds(0, RC)], b, sem).wait()

        def wait_wr(b, sem):
            pltpu.make_async_copy(b, dst.at[pl.ds(0, RC)], sem).wait()

        rd(0, b0, gs0)

        def body(k, _):
            c0 = 2 * k
            c1 = 2 * k + 1
            wait_rd(b0, gs0)

            @pl.when(k > 0)
            def _():
                wait_wr(b2, ss1)

            rd(c1, b2, gs1)
            wr(c0, b0, ss0)
            zr(c0)
            wait_rd(b2, gs1)
            wait_wr(b0, ss0)

            @pl.when(k + 1 < NRC // 2)
            def _():
                rd(c0 + 2, b0, gs0)

            wr(c1, b2, ss1)
            zr(c1)
            return 0
        lax.fori_loop(0, NRC // 2, body, 0)
        wait_wr(b2, ss1)
        for k in range(NRC):
            pltpu.make_async_copy(b1, acc.at[pl.ds(0, RC)], gs2).wait()

    def final():
        def body(k, _):
            lro = lrow(k)
            ro = grow(k)
            pltpu.async_copy(acc.at[pl.ds(lro, RC)], b0, gs0)
            pltpu.async_copy(ego.at[pl.ds(ro, RC)], b1, gs1)
            pltpu.async_copy(y1.at[pl.ds(ro, RC)], b2, gs2)
            pltpu.async_copy(y2.at[pl.ds(ro, RC)], b3, gs3)
            pltpu.make_async_copy(acc.at[pl.ds(0, RC)], b0, gs0).wait()
            pltpu.make_async_copy(ego.at[pl.ds(0, RC)], b1, gs1).wait()
            pltpu.make_async_copy(y1.at[pl.ds(0, RC)], b2, gs2).wait()
            pltpu.make_async_copy(y2.at[pl.ds(0, RC)], b3, gs3).wait()

            def add_body(i, _):
                for g in range(NG):
                    sl = pl.ds(g * LANES, LANES)
                    b0[i, sl] = ((b0[i, sl] + b1[i, sl])
                                 + (b2[i, sl] + b3[i, sl])) * 0.25
                return 0
            lax.fori_loop(0, RC, add_body, 0)
            pltpu.sync_copy(b0, out.at[pl.ds(ro, RC)])
            return 0
        lax.fori_loop(0, NRC, body, 0)

    spmm(ego)
    plsc.subcore_barrier()
    drain(y1)
    plsc.subcore_barrier()
    spmm(y1)
    plsc.subcore_barrier()
    drain(y2)
    plsc.subcore_barrier()
    spmm(y2)
    plsc.subcore_barrier()
    final()


_gcn = functools.partial(
    pl.kernel,
    out_type=[jax.ShapeDtypeStruct((2 * NNP, DD), jnp.float32)] * 3,
    mesh=plsc.VectorSubcoreMesh(core_axis_name="c", subcore_axis_name="s"),
    scratch_types=[
        pltpu.VMEM_SHARED((NNP, DD), jnp.float32),
        pltpu.VMEM((2 * GE,), jnp.int32),
        pltpu.VMEM((2 * GRP, CH), jnp.int32),
        pltpu.VMEM((2 * GE,), jnp.float32),
        pltpu.VMEM((RC, DD), jnp.float32),
        pltpu.VMEM((RC, DD), jnp.float32),
        pltpu.VMEM((RC, DD), jnp.float32),
        pltpu.VMEM((RC, DD), jnp.float32),
        pltpu.SemaphoreType.DMA,
        pltpu.SemaphoreType.DMA,
        pltpu.SemaphoreType.DMA,
        pltpu.SemaphoreType.DMA,
        pltpu.SemaphoreType.DMA,
        pltpu.SemaphoreType.DMA,
        pltpu.SemaphoreType.DMA,
        pltpu.SemaphoreType.DMA,
        pltpu.SemaphoreType.DMA,
        pltpu.SemaphoreType.DMA,
    ],
)(_gcn_body)


def kernel(user_emb, item_emb, user_emb_implict, item_emb_implict,
           adj_row, adj_col, adj_val, adj_imp_row, adj_imp_col, adj_imp_val):
    npad = NNP - NN
    epad = EEP - EE
    ego = jnp.concatenate([
        jnp.pad(jnp.concatenate([user_emb, item_emb], axis=0),
                ((0, npad), (0, 0))),
        jnp.pad(jnp.concatenate([user_emb_implict, item_emb_implict], axis=0),
                ((0, npad), (0, 0))),
    ])
    pad_idx = jnp.arange(epad, dtype=jnp.int32) % NN
    col = jnp.concatenate([adj_col, pad_idx,
                           adj_imp_col + NNP, pad_idx + NNP])
    row = jnp.concatenate([adj_row, pad_idx,
                           adj_imp_row, pad_idx]).reshape(2 * NCHP, CH)
    val = jnp.concatenate([jnp.pad(adj_val, (0, epad)),
                           jnp.pad(adj_imp_val, (0, epad))])
    out, _, _ = _gcn(ego, col, row, val)
    return (out[:USERS], out[USERS:NN],
            out[NNP:NNP + USERS], out[NNP + USERS:NNP + NN])


# speedup vs baseline: 8.6913x; 1.0934x over previous
import functools

import jax
import jax.numpy as jnp
from jax import lax
from jax.experimental import pallas as pl
from jax.experimental.pallas import tpu as pltpu
from jax.experimental.pallas import tpu_sc as plsc

USERS = 4000
ITEMS = 6000
NN = USERS + ITEMS
DD = 128
EE = 320000
NT = 16
CH = 64
NCHP = 5120
EEP = NCHP * CH
CPT = NCHP // NT
GRP = 32
NGRP = CPT // GRP
GE = GRP * CH
NNP = 10240
RTP = NNP // NT
RC = 64
NRC = RTP // RC
LANES = 16
NG = DD // LANES


def _gcn_body(ego, col, rowr, val, out, y1, y2,
              acc, colg, rowg, valg, b0, b1, b2, b3,
              gs0, gs1, gs2, gs3, ss0, ss1, ss2, ss3, ms0, ms1):
    c = lax.axis_index("c")
    tile = lax.axis_index("s")
    rbase = pl.multiple_of(c * NNP + tile * RTP, RTP)
    cbase = pl.multiple_of(c * NCHP + tile * CPT, CPT)
    bufs = (b0, b1, b2, b3)
    gsems = (gs0, gs1, gs2, gs3)
    ssems = (ss0, ss1, ss2, ss3)

    def zero_buf(b):
        def zb(i, _):
            for g in range(NG):
                b[i, pl.ds(g * LANES, LANES)] = jnp.zeros((LANES,),
                                                          jnp.float32)
            return 0
        lax.fori_loop(0, RC, zb, 0)

    def lrow(k):
        return pl.multiple_of(tile * RTP + k * RC, RC)

    def grow(k):
        return pl.multiple_of(rbase + k * RC, RC)

    zero_buf(b3)
    for k in range(NRC):
        pltpu.async_copy(b3, acc.at[pl.ds(lrow(k), RC)], gs2)
    for k in range(NRC):
        pltpu.make_async_copy(b3, acc.at[pl.ds(0, RC)], gs2).wait()
    plsc.subcore_barrier()

    def spmm(src):
        def start_g(ck, b, gs, boff):
            pltpu.async_copy(src.at[colg.at[pl.ds(boff + ck * CH, CH)]],
                             b, gs)

        def wait_g(b, gs):
            pltpu.make_async_copy(src.at[pl.ds(0, CH)], b, gs).wait()

        def start_s(ck, b, ss, goff):
            pltpu.async_copy(b, acc.at[rowg.at[goff + ck]], ss, add=True)

        def wait_s(b, ss):
            pltpu.make_async_copy(b, acc.at[pl.ds(0, CH)], ss).wait()

        def mult(ck, b, boff):
            def mul_body(eb, _):
                vv = valg[pl.ds(boff + ck * CH + eb * LANES, LANES)]
                for i in range(LANES):
                    sv = vv[i]
                    e = eb * LANES + i
                    for g in range(NG):
                        sl = pl.ds(g * LANES, LANES)
                        b[e, sl] = b[e, sl] * sv
                return 0
            lax.fori_loop(0, CH // LANES, mul_body, 0)

        def load_meta(grp, boff, goff, sem):
            gchunk = pl.multiple_of(cbase + grp * GRP, GRP)
            gedge = pl.multiple_of(gchunk * CH, GE)
            pltpu.async_copy(col.at[pl.ds(gedge, GE)],
                             colg.at[pl.ds(boff, GE)], sem)
            pltpu.async_copy(val.at[pl.ds(gedge, GE)],
                             valg.at[pl.ds(boff, GE)], sem)
            pltpu.async_copy(rowr.at[pl.ds(gchunk, GRP)],
                             rowg.at[pl.ds(goff, GRP)], sem)

        def wait_meta(sem):
            pltpu.make_async_copy(col.at[pl.ds(0, GE)],
                                  colg.at[pl.ds(0, GE)], sem).wait()
            pltpu.make_async_copy(val.at[pl.ds(0, GE)],
                                  valg.at[pl.ds(0, GE)], sem).wait()
            pltpu.make_async_copy(rowr.at[pl.ds(0, GRP)],
                                  rowg.at[pl.ds(0, GRP)], sem).wait()

        load_meta(0, 0, 0, ms0)
        wait_meta(ms0)
        load_meta(1, GE, GRP, ms1)

        def group_body(grp, _):
            parity = lax.rem(grp, 2)
            boff = pl.multiple_of(parity * GE, GE)
            goff = pl.multiple_of(parity * GRP, GRP)

            start_g(0, bufs[0], gsems[0], boff)
            start_g(1, bufs[1], gsems[1], boff)

            def quad_body(k, _):
                for j in range(4):
                    ck = k * 4 + j
                    nb = (j + 2) % 4
                    wait_g(bufs[j], gsems[j])

                    @pl.when(ck >= 2)
                    def _():
                        wait_s(bufs[nb], ssems[nb])

                    @pl.when(ck + 2 < GRP)
                    def _():
                        start_g(ck + 2, bufs[nb], gsems[nb], boff)

                    mult(ck, bufs[j], boff)
                    start_s(ck, bufs[j], ssems[j], goff)
                return 0
            lax.fori_loop(0, GRP // 4, quad_body, 0)

            wait_s(bufs[2], ssems[2])
            wait_s(bufs[3], ssems[3])

            @pl.when(grp + 2 < NGRP)
            def _():
                @pl.when(parity == 0)
                def _():
                    load_meta(grp + 2, 0, 0, ms0)

                @pl.when(parity == 1)
                def _():
                    load_meta(grp + 2, GE, GRP, ms1)

            @pl.when(grp + 1 < NGRP)
            def _():
                @pl.when(parity == 0)
                def _():
                    wait_meta(ms1)

                @pl.when(parity == 1)
                def _():
                    wait_meta(ms0)
            return 0
        lax.fori_loop(0, NGRP, group_body, 0)

    def drain(dst):
        zero_buf(b1)

        def rd(k, b, sem):
            pltpu.async_copy(acc.at[pl.ds(lrow(k), RC)], b, sem)

        def wr(k, b, sem):
            pltpu.async_copy(b, dst.at[pl.ds(grow(k), RC)], sem)

        def zr(k):
            pltpu.async_copy(b1, acc.at[pl.ds(lrow(k), RC)], gs2)

        def wait_rd(b, sem):
            pltpu.make_async_copy(acc.at[pl.ds(0, RC)], b, sem).wait()

        def wait_wr(b, sem):
            pltpu.make_async_copy(b, dst.at[pl.ds(0, RC)], sem).wait()

        rd(0, b0, gs0)

        def body(k, _):
            c0 = 2 * k
            c1 = 2 * k + 1
            wait_rd(b0, gs0)

            @pl.when(k > 0)
            def _():
                wait_wr(b2, ss1)

            rd(c1, b2, gs1)
            wr(c0, b0, ss0)
            zr(c0)
            wait_rd(b2, gs1)
            wait_wr(b0, ss0)

            @pl.when(k + 1 < NRC // 2)
            def _():
                rd(c0 + 2, b0, gs0)

            wr(c1, b2, ss1)
            zr(c1)
            return 0
        lax.fori_loop(0, NRC // 2, body, 0)
        wait_wr(b2, ss1)
        for k in range(NRC):
            pltpu.make_async_copy(b1, acc.at[pl.ds(0, RC)], gs2).wait()

    def final():
        def body(k, _):
            lro = lrow(k)
            ro = grow(k)
            pltpu.async_copy(acc.at[pl.ds(lro, RC)], b0, gs0)
            pltpu.async_copy(ego.at[pl.ds(ro, RC)], b1, gs1)
            pltpu.async_copy(y1.at[pl.ds(ro, RC)], b2, gs2)
            pltpu.async_copy(y2.at[pl.ds(ro, RC)], b3, gs3)
            pltpu.make_async_copy(acc.at[pl.ds(0, RC)], b0, gs0).wait()
            pltpu.make_async_copy(ego.at[pl.ds(0, RC)], b1, gs1).wait()
            pltpu.make_async_copy(y1.at[pl.ds(0, RC)], b2, gs2).wait()
            pltpu.make_async_copy(y2.at[pl.ds(0, RC)], b3, gs3).wait()

            def add_body(i, _):
                for g in range(NG):
                    sl = pl.ds(g * LANES, LANES)
                    b0[i, sl] = ((b0[i, sl] + b1[i, sl])
                                 + (b2[i, sl] + b3[i, sl])) * 0.25
                return 0
            lax.fori_loop(0, RC, add_body, 0)
            pltpu.sync_copy(b0, out.at[pl.ds(ro, RC)])
            return 0
        lax.fori_loop(0, NRC, body, 0)

    spmm(ego)
    plsc.subcore_barrier()
    drain(y1)
    plsc.subcore_barrier()
    spmm(y1)
    plsc.subcore_barrier()
    drain(y2)
    plsc.subcore_barrier()
    spmm(y2)
    plsc.subcore_barrier()
    final()


_gcn = functools.partial(
    pl.kernel,
    out_type=[jax.ShapeDtypeStruct((2 * NNP, DD), jnp.float32)] * 3,
    mesh=plsc.VectorSubcoreMesh(core_axis_name="c", subcore_axis_name="s"),
    scratch_types=[
        pltpu.VMEM_SHARED((NNP, DD), jnp.float32),
        pltpu.VMEM((2 * GE,), jnp.int32),
        pltpu.VMEM((2 * GRP, CH), jnp.int32),
        pltpu.VMEM((2 * GE,), jnp.float32),
        pltpu.VMEM((RC, DD), jnp.float32),
        pltpu.VMEM((RC, DD), jnp.float32),
        pltpu.VMEM((RC, DD), jnp.float32),
        pltpu.VMEM((RC, DD), jnp.float32),
        pltpu.SemaphoreType.DMA,
        pltpu.SemaphoreType.DMA,
        pltpu.SemaphoreType.DMA,
        pltpu.SemaphoreType.DMA,
        pltpu.SemaphoreType.DMA,
        pltpu.SemaphoreType.DMA,
        pltpu.SemaphoreType.DMA,
        pltpu.SemaphoreType.DMA,
        pltpu.SemaphoreType.DMA,
        pltpu.SemaphoreType.DMA,
    ],
)(_gcn_body)


def kernel(user_emb, item_emb, user_emb_implict, item_emb_implict,
           adj_row, adj_col, adj_val, adj_imp_row, adj_imp_col, adj_imp_val):
    npad = NNP - NN
    epad = EEP - EE
    ego = jnp.concatenate([
        jnp.pad(jnp.concatenate([user_emb, item_emb], axis=0),
                ((0, npad), (0, 0))),
        jnp.pad(jnp.concatenate([user_emb_implict, item_emb_implict], axis=0),
                ((0, npad), (0, 0))),
    ])
    pad_idx = jnp.arange(epad, dtype=jnp.int32) % NN
    col = jnp.concatenate([adj_col, pad_idx,
                           adj_imp_col + NNP, pad_idx + NNP])
    row = jnp.concatenate([adj_row, pad_idx,
                           adj_imp_row, pad_idx]).reshape(2 * NCHP, CH)
    val = jnp.concatenate([jnp.pad(adj_val, (0, epad)),
                           jnp.pad(adj_imp_val, (0, epad))])
    out, _, _ = _gcn(ego, col, row, val)
    return (out[:USERS], out[USERS:NN],
            out[NNP:NNP + USERS], out[NNP + USERS:NNP + NN])
